# baseline (speedup 1.0000x reference)
import jax
import jax.numpy as jnp
from jax import lax
from jax.experimental import pallas as pl
from jax.experimental.pallas import tpu as pltpu
from jax.experimental.pallas import tpu_sc as plsc

_L = 16
_NC, _NS = 2, 16
_NW = _NC * _NS
_CR = 8
_UNROLL = 4
_NACC = 4

_NBUF = 2
_SC_ROWS = 6656
_TC_BLK = 256


def _accum_chunk(ncols, pbuf, tbuf, acc):
    def row_loop(r, acc):
        def inner(k, acc):
            accl = list(acc)
            for u in range(_UNROLL):
                off = (k * _UNROLL + u) * _L
                p = pbuf[r, pl.ds(off, _L)]
                t = tbuf[r, pl.ds(off, _L)]
                m0 = t == 0.0
                d = jnp.where(m0, 0.0, p - t)
                a = u % _NACC
                accl[a] = accl[a] + d * d
                accl[_NACC + a] = accl[_NACC + a] + jnp.where(m0, 1.0, 0.0)
            return tuple(accl)

        return lax.fori_loop(0, ncols // _L // _UNROLL, inner, acc)

    return lax.fori_loop(0, _CR, row_loop, acc)


def _make_sc_body(row0, rows_per_w, ncols, nchunk):
    def body(p_hbm, t_hbm, out_hbm, *scratch):
        pbufs = scratch[:_NBUF]
        tbufs = scratch[_NBUF:2 * _NBUF]
        ob = scratch[2 * _NBUF]
        sems = scratch[2 * _NBUF + 1:]
        wid = lax.axis_index("s") * _NC + lax.axis_index("c")
        base = row0 + wid * rows_per_w
        bufs = tuple((pbufs[b], tbufs[b], sems[b]) for b in range(_NBUF))

        def start(i, pbuf, tbuf, sem):
            row = base + i * _CR
            pltpu.make_async_copy(p_hbm.at[pl.ds(row, _CR), :], pbuf, sem).start()
            pltpu.make_async_copy(t_hbm.at[pl.ds(row, _CR), :], tbuf, sem).start()

        def wait(i, pbuf, tbuf, sem):
            row = base + i * _CR
            pltpu.make_async_copy(p_hbm.at[pl.ds(row, _CR), :], pbuf, sem).wait()
            pltpu.make_async_copy(t_hbm.at[pl.ds(row, _CR), :], tbuf, sem).wait()

        for b in range(_NBUF):
            start(b, *bufs[b])
        acc = tuple(jnp.zeros((_L,), jnp.float32) for _ in range(2 * _NACC))

        def outer(q, acc):
            i0 = _NBUF * q
            for b in range(_NBUF):
                wait(i0 + b, *bufs[b])
                acc = _accum_chunk(ncols, bufs[b][0], bufs[b][1], acc)
                start(i0 + b + _NBUF, *bufs[b])
            return acc

        acc = lax.fori_loop(0, nchunk // _NBUF - 1, outer, acc)
        for b in range(_NBUF):
            wait(nchunk - _NBUF + b, *bufs[b])
            acc = _accum_chunk(ncols, bufs[b][0], bufs[b][1], acc)

        s = (acc[0] + acc[1]) + (acc[2] + acc[3])
        nzero = (acc[4] + acc[5]) + (acc[6] + acc[7])
        ob[pl.ds(0, _L)] = s
        ob[pl.ds(_L, _L)] = nzero
        pltpu.sync_copy(ob, out_hbm.at[pl.ds(wid * 2 * _L, 2 * _L)])

    return body


def _sc_partials(y_pred, y_true, row0, nrows_sc):
    ncols = y_pred.shape[1]
    rows_per_w = nrows_sc // _NW
    nchunk = rows_per_w // _CR
    f = pl.kernel(
        _make_sc_body(row0, rows_per_w, ncols, nchunk),
        out_type=jax.ShapeDtypeStruct((_NW * 2 * _L,), jnp.float32),
        mesh=plsc.VectorSubcoreMesh(core_axis_name="c", subcore_axis_name="s"),
        scratch_types=(
            [pltpu.VMEM((_CR, ncols), jnp.float32) for _ in range(2 * _NBUF)]
            + [pltpu.VMEM((2 * _L,), jnp.float32)]
            + [pltpu.SemaphoreType.DMA for _ in range(_NBUF)]
        ),
    )
    return f(y_pred, y_true)


def _tc_body(nblocks, p_ref, t_ref, out_ref, acc_ref):
    i = pl.program_id(0)

    @pl.when(i == 0)
    def _init():
        acc_ref[0, 0] = 0.0
        acc_ref[0, 1] = 0.0

    p = p_ref[...]
    t = t_ref[...]
    mask = t != 0.0
    d = jnp.where(mask, p - t, 0.0)
    acc_ref[0, 0] += jnp.sum(d * d)
    acc_ref[0, 1] += jnp.sum(mask.astype(jnp.float32))

    @pl.when(i == nblocks - 1)
    def _fin():
        out_ref[0, 0] = acc_ref[0, 0]
        out_ref[0, 1] = acc_ref[0, 1]


def _tc_partials(y_pred, y_true, nrows_tc):
    ncols = y_pred.shape[1]
    nblocks = nrows_tc // _TC_BLK
    return pl.pallas_call(
        lambda p, t, o, a: _tc_body(nblocks, p, t, o, a),
        grid=(nblocks,),
        in_specs=[
            pl.BlockSpec((_TC_BLK, ncols), lambda i: (i, 0)),
            pl.BlockSpec((_TC_BLK, ncols), lambda i: (i, 0)),
        ],
        out_specs=pl.BlockSpec(memory_space=pltpu.SMEM),
        out_shape=jax.ShapeDtypeStruct((1, 2), jnp.float32),
        scratch_shapes=[pltpu.SMEM((1, 2), jnp.float32)],
    )(y_pred, y_true)


def kernel(y_pred, y_true):
    nrows, ncols = y_pred.shape
    nrows_tc = nrows - _SC_ROWS

    sc_out = _sc_partials(y_pred, y_true, nrows_tc, _SC_ROWS)
    tc_out = _tc_partials(y_pred, y_true, nrows_tc)

    r = sc_out.reshape(_NW, 2, _L)
    ssum = jnp.sum(r[:, 0, :]) + tc_out[0, 0]
    sc_cnt = float(_SC_ROWS * ncols) - jnp.sum(r[:, 1, :])
    cnt = sc_cnt + tc_out[0, 1]
    return jnp.where(cnt > 0.0, ssum / jnp.maximum(cnt, 1.0), 0.0)


# speedup vs baseline: 1.0166x; 1.0166x over previous
import jax
import jax.numpy as jnp
from jax import lax
from jax.experimental import pallas as pl
from jax.experimental.pallas import tpu as pltpu
from jax.experimental.pallas import tpu_sc as plsc

_L = 16
_NC, _NS = 2, 16
_NW = _NC * _NS
_CR = 8
_UNROLL = 4
_NACC = 4

_NBUF = 2
_SC_ROWS = 6656
_TC_BLK = 1216


def _accum_chunk(ncols, pbuf, tbuf, acc):
    def row_loop(r, acc):
        def inner(k, acc):
            accl = list(acc)
            for u in range(_UNROLL):
                off = (k * _UNROLL + u) * _L
                p = pbuf[r, pl.ds(off, _L)]
                t = tbuf[r, pl.ds(off, _L)]
                m0 = t == 0.0
                d = jnp.where(m0, 0.0, p - t)
                a = u % _NACC
                accl[a] = accl[a] + d * d
                accl[_NACC + a] = accl[_NACC + a] + jnp.where(m0, 1.0, 0.0)
            return tuple(accl)

        return lax.fori_loop(0, ncols // _L // _UNROLL, inner, acc)

    return lax.fori_loop(0, _CR, row_loop, acc)


def _make_sc_body(row0, rows_per_w, ncols, nchunk):
    def body(p_hbm, t_hbm, out_hbm, *scratch):
        pbufs = scratch[:_NBUF]
        tbufs = scratch[_NBUF:2 * _NBUF]
        ob = scratch[2 * _NBUF]
        sems = scratch[2 * _NBUF + 1:]
        wid = lax.axis_index("s") * _NC + lax.axis_index("c")
        base = row0 + wid * rows_per_w
        bufs = tuple((pbufs[b], tbufs[b], sems[b]) for b in range(_NBUF))

        def start(i, pbuf, tbuf, sem):
            row = base + i * _CR
            pltpu.make_async_copy(p_hbm.at[pl.ds(row, _CR), :], pbuf, sem).start()
            pltpu.make_async_copy(t_hbm.at[pl.ds(row, _CR), :], tbuf, sem).start()

        def wait(i, pbuf, tbuf, sem):
            row = base + i * _CR
            pltpu.make_async_copy(p_hbm.at[pl.ds(row, _CR), :], pbuf, sem).wait()
            pltpu.make_async_copy(t_hbm.at[pl.ds(row, _CR), :], tbuf, sem).wait()

        for b in range(_NBUF):
            start(b, *bufs[b])
        acc = tuple(jnp.zeros((_L,), jnp.float32) for _ in range(2 * _NACC))

        def outer(q, acc):
            i0 = _NBUF * q
            for b in range(_NBUF):
                wait(i0 + b, *bufs[b])
                acc = _accum_chunk(ncols, bufs[b][0], bufs[b][1], acc)
                start(i0 + b + _NBUF, *bufs[b])
            return acc

        acc = lax.fori_loop(0, nchunk // _NBUF - 1, outer, acc)
        for b in range(_NBUF):
            wait(nchunk - _NBUF + b, *bufs[b])
            acc = _accum_chunk(ncols, bufs[b][0], bufs[b][1], acc)

        s = (acc[0] + acc[1]) + (acc[2] + acc[3])
        nzero = (acc[4] + acc[5]) + (acc[6] + acc[7])
        ob[pl.ds(0, _L)] = s
        ob[pl.ds(_L, _L)] = nzero
        pltpu.sync_copy(ob, out_hbm.at[pl.ds(wid * 2 * _L, 2 * _L)])

    return body


def _sc_partials(y_pred, y_true, row0, nrows_sc):
    ncols = y_pred.shape[1]
    rows_per_w = nrows_sc // _NW
    nchunk = rows_per_w // _CR
    f = pl.kernel(
        _make_sc_body(row0, rows_per_w, ncols, nchunk),
        out_type=jax.ShapeDtypeStruct((_NW * 2 * _L,), jnp.float32),
        mesh=plsc.VectorSubcoreMesh(core_axis_name="c", subcore_axis_name="s"),
        scratch_types=(
            [pltpu.VMEM((_CR, ncols), jnp.float32) for _ in range(2 * _NBUF)]
            + [pltpu.VMEM((2 * _L,), jnp.float32)]
            + [pltpu.SemaphoreType.DMA for _ in range(_NBUF)]
        ),
    )
    return f(y_pred, y_true)


def _tc_body(nblocks, p_ref, t_ref, out_ref, acc_ref):
    i = pl.program_id(0)

    @pl.when(i == 0)
    def _init():
        acc_ref[0, 0] = 0.0
        acc_ref[0, 1] = 0.0

    p = p_ref[...]
    t = t_ref[...]
    mask = t != 0.0
    d = jnp.where(mask, p - t, 0.0)
    acc_ref[0, 0] += jnp.sum(d * d)
    acc_ref[0, 1] += jnp.sum(mask.astype(jnp.float32))

    @pl.when(i == nblocks - 1)
    def _fin():
        out_ref[0, 0] = acc_ref[0, 0]
        out_ref[0, 1] = acc_ref[0, 1]


def _tc_partials(y_pred, y_true, nrows_tc):
    ncols = y_pred.shape[1]
    nblocks = nrows_tc // _TC_BLK
    return pl.pallas_call(
        lambda p, t, o, a: _tc_body(nblocks, p, t, o, a),
        grid=(nblocks,),
        in_specs=[
            pl.BlockSpec((_TC_BLK, ncols), lambda i: (i, 0)),
            pl.BlockSpec((_TC_BLK, ncols), lambda i: (i, 0)),
        ],
        out_specs=pl.BlockSpec(memory_space=pltpu.SMEM),
        out_shape=jax.ShapeDtypeStruct((1, 2), jnp.float32),
        scratch_shapes=[pltpu.SMEM((1, 2), jnp.float32)],
    )(y_pred, y_true)


def kernel(y_pred, y_true):
    nrows, ncols = y_pred.shape
    nrows_tc = nrows - _SC_ROWS

    sc_out = _sc_partials(y_pred, y_true, nrows_tc, _SC_ROWS)
    tc_out = _tc_partials(y_pred, y_true, nrows_tc)

    r = sc_out.reshape(_NW, 2, _L)
    ssum = jnp.sum(r[:, 0, :]) + tc_out[0, 0]
    sc_cnt = float(_SC_ROWS * ncols) - jnp.sum(r[:, 1, :])
    cnt = sc_cnt + tc_out[0, 1]
    return jnp.where(cnt > 0.0, ssum / jnp.maximum(cnt, 1.0), 0.0)


# speedup vs baseline: 1.0292x; 1.0124x over previous
import jax
import jax.numpy as jnp
from jax import lax
from jax.experimental import pallas as pl
from jax.experimental.pallas import tpu as pltpu
from jax.experimental.pallas import tpu_sc as plsc

_L = 16
_NC, _NS = 2, 16
_NW = _NC * _NS
_CR = 8
_UNROLL = 4
_NACC = 4

_NBUF = 2
_SC_ROWS = 6656
_TC_BLK = 512


def _accum_chunk(ncols, pbuf, tbuf, acc):
    def row_loop(r, acc):
        def inner(k, acc):
            accl = list(acc)
            for u in range(_UNROLL):
                off = (k * _UNROLL + u) * _L
                p = pbuf[r, pl.ds(off, _L)]
                t = tbuf[r, pl.ds(off, _L)]
                m0 = t == 0.0
                d = jnp.where(m0, 0.0, p - t)
                a = u % _NACC
                accl[a] = accl[a] + d * d
                accl[_NACC + a] = accl[_NACC + a] + jnp.where(m0, 1.0, 0.0)
            return tuple(accl)

        return lax.fori_loop(0, ncols // _L // _UNROLL, inner, acc)

    return lax.fori_loop(0, _CR, row_loop, acc)


def _make_sc_body(row0, rows_per_w, ncols, nchunk):
    def body(p_hbm, t_hbm, out_hbm, *scratch):
        pbufs = scratch[:_NBUF]
        tbufs = scratch[_NBUF:2 * _NBUF]
        ob = scratch[2 * _NBUF]
        sems = scratch[2 * _NBUF + 1:]
        wid = lax.axis_index("s") * _NC + lax.axis_index("c")
        base = row0 + wid * rows_per_w
        bufs = tuple((pbufs[b], tbufs[b], sems[b]) for b in range(_NBUF))

        def start(i, pbuf, tbuf, sem):
            row = base + i * _CR
            pltpu.make_async_copy(p_hbm.at[pl.ds(row, _CR), :], pbuf, sem).start()
            pltpu.make_async_copy(t_hbm.at[pl.ds(row, _CR), :], tbuf, sem).start()

        def wait(i, pbuf, tbuf, sem):
            row = base + i * _CR
            pltpu.make_async_copy(p_hbm.at[pl.ds(row, _CR), :], pbuf, sem).wait()
            pltpu.make_async_copy(t_hbm.at[pl.ds(row, _CR), :], tbuf, sem).wait()

        for b in range(_NBUF):
            start(b, *bufs[b])
        acc = tuple(jnp.zeros((_L,), jnp.float32) for _ in range(2 * _NACC))

        def outer(q, acc):
            i0 = _NBUF * q
            for b in range(_NBUF):
                wait(i0 + b, *bufs[b])
                acc = _accum_chunk(ncols, bufs[b][0], bufs[b][1], acc)
                start(i0 + b + _NBUF, *bufs[b])
            return acc

        acc = lax.fori_loop(0, nchunk // _NBUF - 1, outer, acc)
        for b in range(_NBUF):
            wait(nchunk - _NBUF + b, *bufs[b])
            acc = _accum_chunk(ncols, bufs[b][0], bufs[b][1], acc)

        s = (acc[0] + acc[1]) + (acc[2] + acc[3])
        nzero = (acc[4] + acc[5]) + (acc[6] + acc[7])
        ob[pl.ds(0, _L)] = s
        ob[pl.ds(_L, _L)] = nzero
        pltpu.sync_copy(ob, out_hbm.at[pl.ds(wid * 2 * _L, 2 * _L)])

    return body


def _sc_partials(y_pred, y_true, row0, nrows_sc):
    ncols = y_pred.shape[1]
    rows_per_w = nrows_sc // _NW
    nchunk = rows_per_w // _CR
    f = pl.kernel(
        _make_sc_body(row0, rows_per_w, ncols, nchunk),
        out_type=jax.ShapeDtypeStruct((_NW * 2 * _L,), jnp.float32),
        mesh=plsc.VectorSubcoreMesh(core_axis_name="c", subcore_axis_name="s"),
        scratch_types=(
            [pltpu.VMEM((_CR, ncols), jnp.float32) for _ in range(2 * _NBUF)]
            + [pltpu.VMEM((2 * _L,), jnp.float32)]
            + [pltpu.SemaphoreType.DMA for _ in range(_NBUF)]
        ),
    )
    return f(y_pred, y_true)


def _tc_body(nblocks, p_ref, t_ref, out_ref, acc_ref):
    i = pl.program_id(0)

    @pl.when(i == 0)
    def _init():
        acc_ref[0, 0] = 0.0
        acc_ref[0, 1] = 0.0

    p = p_ref[...]
    t = t_ref[...]
    mask = t != 0.0
    d = jnp.where(mask, p - t, 0.0)
    acc_ref[0, 0] += jnp.sum(d * d)
    acc_ref[0, 1] += jnp.sum(mask.astype(jnp.float32))

    @pl.when(i == nblocks - 1)
    def _fin():
        out_ref[0, 0] = acc_ref[0, 0]
        out_ref[0, 1] = acc_ref[0, 1]


def _tc_partials(y_pred, y_true, nrows_tc):
    ncols = y_pred.shape[1]
    nblocks = nrows_tc // _TC_BLK
    return pl.pallas_call(
        lambda p, t, o, a: _tc_body(nblocks, p, t, o, a),
        grid=(nblocks,),
        in_specs=[
            pl.BlockSpec((_TC_BLK, ncols), lambda i: (i, 0)),
            pl.BlockSpec((_TC_BLK, ncols), lambda i: (i, 0)),
        ],
        out_specs=pl.BlockSpec(memory_space=pltpu.SMEM),
        out_shape=jax.ShapeDtypeStruct((1, 2), jnp.float32),
        scratch_shapes=[pltpu.SMEM((1, 2), jnp.float32)],
    )(y_pred, y_true)


def kernel(y_pred, y_true):
    nrows, ncols = y_pred.shape
    nrows_tc = nrows - _SC_ROWS

    sc_out = _sc_partials(y_pred, y_true, nrows_tc, _SC_ROWS)
    tc_out = _tc_partials(y_pred, y_true, nrows_tc)

    r = sc_out.reshape(_NW, 2, _L)
    ssum = jnp.sum(r[:, 0, :]) + tc_out[0, 0]
    sc_cnt = float(_SC_ROWS * ncols) - jnp.sum(r[:, 1, :])
    cnt = sc_cnt + tc_out[0, 1]
    return jnp.where(cnt > 0.0, ssum / jnp.maximum(cnt, 1.0), 0.0)


# speedup vs baseline: 1.0294x; 1.0002x over previous
import jax
import jax.numpy as jnp
from jax import lax
from jax.experimental import pallas as pl
from jax.experimental.pallas import tpu as pltpu
from jax.experimental.pallas import tpu_sc as plsc

_L = 16
_NC, _NS = 2, 16
_NW = _NC * _NS
_CR = 8
_UNROLL = 4
_NACC = 4

_NBUF = 2
_SC_ROWS = 6656
_TC_BLK = 512


def _accum_chunk(ncols, pbuf, tbuf, acc):
    def row_loop(r, acc):
        def inner(k, acc):
            accl = list(acc)
            for u in range(_UNROLL):
                off = (k * _UNROLL + u) * _L
                p = pbuf[r, pl.ds(off, _L)]
                t = tbuf[r, pl.ds(off, _L)]
                m0 = t == 0.0
                d = jnp.where(m0, 0.0, p - t)
                a = u % _NACC
                accl[a] = accl[a] + d * d
                accl[_NACC + a] = accl[_NACC + a] + jnp.where(m0, 1.0, 0.0)
            return tuple(accl)

        return lax.fori_loop(0, ncols // _L // _UNROLL, inner, acc)

    return lax.fori_loop(0, _CR, row_loop, acc)


def _make_sc_body(row0, rows_per_w, ncols, nchunk):
    def body(p_hbm, t_hbm, out_hbm, *scratch):
        pbufs = scratch[:_NBUF]
        tbufs = scratch[_NBUF:2 * _NBUF]
        ob = scratch[2 * _NBUF]
        sems = scratch[2 * _NBUF + 1:]
        wid = lax.axis_index("s") * _NC + lax.axis_index("c")
        base = row0 + wid * rows_per_w
        bufs = tuple((pbufs[b], tbufs[b], sems[b]) for b in range(_NBUF))

        def start(i, pbuf, tbuf, sem):
            row = base + i * _CR
            pltpu.make_async_copy(p_hbm.at[pl.ds(row, _CR), :], pbuf, sem).start()
            pltpu.make_async_copy(t_hbm.at[pl.ds(row, _CR), :], tbuf, sem).start()

        def wait(i, pbuf, tbuf, sem):
            row = base + i * _CR
            pltpu.make_async_copy(p_hbm.at[pl.ds(row, _CR), :], pbuf, sem).wait()
            pltpu.make_async_copy(t_hbm.at[pl.ds(row, _CR), :], tbuf, sem).wait()

        for b in range(_NBUF):
            start(b, *bufs[b])
        acc = tuple(jnp.zeros((_L,), jnp.float32) for _ in range(2 * _NACC))

        def outer(q, acc):
            i0 = _NBUF * q
            for b in range(_NBUF):
                wait(i0 + b, *bufs[b])
                acc = _accum_chunk(ncols, bufs[b][0], bufs[b][1], acc)
                start(i0 + b + _NBUF, *bufs[b])
            return acc

        acc = lax.fori_loop(0, nchunk // _NBUF - 1, outer, acc)
        for b in range(_NBUF):
            wait(nchunk - _NBUF + b, *bufs[b])
            acc = _accum_chunk(ncols, bufs[b][0], bufs[b][1], acc)

        s = (acc[0] + acc[1]) + (acc[2] + acc[3])
        nzero = (acc[4] + acc[5]) + (acc[6] + acc[7])
        ob[pl.ds(0, _L)] = s
        ob[pl.ds(_L, _L)] = nzero
        pltpu.sync_copy(ob, out_hbm.at[pl.ds(wid * 2 * _L, 2 * _L)])

    return body


def _sc_partials(y_pred, y_true, row0, nrows_sc):
    ncols = y_pred.shape[1]
    rows_per_w = nrows_sc // _NW
    nchunk = rows_per_w // _CR
    f = pl.kernel(
        _make_sc_body(row0, rows_per_w, ncols, nchunk),
        out_type=jax.ShapeDtypeStruct((_NW * 2 * _L,), jnp.float32),
        mesh=plsc.VectorSubcoreMesh(core_axis_name="c", subcore_axis_name="s"),
        scratch_types=(
            [pltpu.VMEM((_CR, ncols), jnp.float32) for _ in range(2 * _NBUF)]
            + [pltpu.VMEM((2 * _L,), jnp.float32)]
            + [pltpu.SemaphoreType.DMA for _ in range(_NBUF)]
        ),
    )
    return f(y_pred, y_true)


def _tc_body(nblocks, p_ref, t_ref, out_ref, acc_ref):
    i = pl.program_id(0)

    @pl.when(i == 0)
    def _init():
        acc_ref[0, 0] = 0.0
        acc_ref[0, 1] = 0.0

    p = p_ref[...]
    t = t_ref[...]
    mask = t != 0.0
    d = jnp.where(mask, p - t, 0.0)
    acc_ref[0, 0] += jnp.sum(d * d)
    acc_ref[0, 1] += jnp.sum(mask.astype(jnp.float32))

    @pl.when(i == nblocks - 1)
    def _fin():
        out_ref[0, 0] = acc_ref[0, 0]
        out_ref[0, 1] = acc_ref[0, 1]


def _tc_partials(y_pred, y_true, nrows_tc):
    ncols = y_pred.shape[1]
    nblocks = nrows_tc // _TC_BLK
    return pl.pallas_call(
        lambda p, t, o, a: _tc_body(nblocks, p, t, o, a),
        grid=(nblocks,),
        in_specs=[
            pl.BlockSpec((_TC_BLK, ncols), lambda i: (i, 0)),
            pl.BlockSpec((_TC_BLK, ncols), lambda i: (i, 0)),
        ],
        out_specs=pl.BlockSpec(memory_space=pltpu.SMEM),
        out_shape=jax.ShapeDtypeStruct((1, 2), jnp.float32),
        scratch_shapes=[pltpu.SMEM((1, 2), jnp.float32)],
    )(y_pred, y_true)


def kernel(y_pred, y_true):
    nrows, ncols = y_pred.shape
    nrows_tc = nrows - _SC_ROWS

    tc_out = _tc_partials(y_pred, y_true, nrows_tc)
    sc_out = _sc_partials(y_pred, y_true, nrows_tc, _SC_ROWS)

    r = sc_out.reshape(_NW, 2, _L)
    ssum = jnp.sum(r[:, 0, :]) + tc_out[0, 0]
    sc_cnt = float(_SC_ROWS * ncols) - jnp.sum(r[:, 1, :])
    cnt = sc_cnt + tc_out[0, 1]
    return jnp.where(cnt > 0.0, ssum / jnp.maximum(cnt, 1.0), 0.0)
